# baseline (speedup 1.0000x reference)
import functools

import jax
import jax.numpy as jnp
from jax import lax
from jax.experimental import pallas as pl
from jax.experimental.pallas import tpu as pltpu
from jax.experimental.pallas import tpu_sc as plsc

N = 10000
NEIGH = 32
IN_DIM = 128
OUT_DIM = 256
MID = 64
K = 15
KP_EXTENT = 1.2
TW = 128
NTBL = N + 8
BN = 400
NBLK = N // BN

_SC = plsc.get_sparse_core_info()
_NC = _SC.num_cores
_NS = _SC.num_subcores
_NW = _NC * _NS
_ROWS = N * NEIGH
_RPW = _ROWS // _NW
_CHUNK = 400
_QROWS = 24


def _leaky(x):
    return jnp.where(x >= 0, x, 0.1 * x)


def _table_kernel(feat_ref, pts_ref, kp_ref, w1_ref, b1_ref, out_ref):
    x = jnp.dot(feat_ref[...], w1_ref[...], preferred_element_type=jnp.float32)
    x = _leaky(x + b1_ref[...])
    pts = pts_ref[...]
    strips = [jnp.broadcast_to(pts[:, j:j + 1], (N, 16)) for j in range(3)]
    pad = jnp.zeros((N, TW - MID - 48), dtype=jnp.float32)
    main = jnp.concatenate([x] + strips + [pad], axis=1)
    out_ref[...] = jnp.concatenate([main, kp_ref[...]], axis=0)


def _build_table(features, points, kp_rows, W1, b1):
    return pl.pallas_call(
        _table_kernel,
        out_shape=jax.ShapeDtypeStruct((NTBL, TW), jnp.float32),
    )(features, points, kp_rows, W1, b1.reshape(1, MID))


def _sc_gather(table, idx_flat):
    mesh = plsc.VectorSubcoreMesh(core_axis_name="c", subcore_axis_name="s")

    @functools.partial(
        pl.kernel,
        mesh=mesh,
        out_type=jax.ShapeDtypeStruct((_ROWS, TW), jnp.float32),
        scratch_types=[
            pltpu.VMEM((_CHUNK,), jnp.int32),
            pltpu.VMEM((_CHUNK, TW), jnp.float32),
            pltpu.VMEM((_QROWS, TW), jnp.float32),
            pltpu.VMEM((8, TW), jnp.float32),
            pltpu.SemaphoreType.DMA,
        ],
    )
    def gather_k(table_hbm, idx_hbm, out_hbm, idx_v, rows_v, q_v, kp_v, sem):
        wid = lax.axis_index("s") * _NC + lax.axis_index("c")
        base = wid * _RPW
        pltpu.sync_copy(table_hbm.at[pl.ds(N, 8)], kp_v)
        kpx = kp_v[0, 0:16]
        kpy = kp_v[0, 16:32]
        kpz = kp_v[0, 32:48]
        inv_ext = 1.0 / KP_EXTENT

        def chunk_body(i, _):
            off = base + i * _CHUNK
            pltpu.sync_copy(idx_hbm.at[pl.ds(off, _CHUNK)], idx_v)
            pltpu.async_copy(table_hbm.at[idx_v], rows_v, sem).wait()
            n0 = pl.multiple_of((off >> 5) & ~7, 8)
            pltpu.sync_copy(table_hbm.at[pl.ds(n0, _QROWS)], q_v)

            def row_body(j, _):
                qi = ((off + j) >> 5) - n0
                px = rows_v[j, pl.ds(MID, 16)]
                py = rows_v[j, pl.ds(MID + 16, 16)]
                pz = rows_v[j, pl.ds(MID + 32, 16)]
                dx = px - q_v[qi, pl.ds(MID, 16)] - kpx
                dy = py - q_v[qi, pl.ds(MID + 16, 16)] - kpy
                dz = pz - q_v[qi, pl.ds(MID + 32, 16)] - kpz
                d2 = dx * dx + dy * dy + dz * dz
                u = lax.bitcast_convert_type(d2, jnp.int32)
                u = 0x5F3759DF - (u >> 1)
                r = lax.bitcast_convert_type(u, jnp.float32)
                h = 0.5 * d2
                r = r * (1.5 - h * r * r)
                r = r * (1.5 - h * r * r)
                r = r * (1.5 - h * r * r)
                w = jnp.maximum(1.0 - (d2 * r) * inv_ext, 0.0)
                rows_v[j, pl.ds(MID, 16)] = w
                return 0

            lax.fori_loop(0, _CHUNK, row_body, 0)
            pltpu.sync_copy(rows_v, out_hbm.at[pl.ds(off, _CHUNK)])
            return 0

        lax.fori_loop(0, _RPW // _CHUNK, chunk_body, 0)

    return gather_k(table, idx_flat)


def _block_kernel(g_ref, feat_ref, wkp_ref, bconv_ref, w2_ref, b2_ref,
                  wsc_ref, bsc_ref, out_ref):
    g = g_ref[...]
    nx = g[:, :MID]
    w = g[:, MID:MID + K]

    wr = w.reshape(BN, NEIGH, K)
    nxr = nx.reshape(BN, NEIGH, MID)
    agg = lax.dot_general(wr, nxr, (((1,), (1,)), ((0,), (0,))),
                          preferred_element_type=jnp.float32)
    agg = agg.reshape(BN, K * MID)

    xkp = jnp.dot(agg, wkp_ref[...], preferred_element_type=jnp.float32)
    x2 = _leaky(xkp + bconv_ref[...])
    x3 = jnp.dot(x2, w2_ref[...], preferred_element_type=jnp.float32) + b2_ref[...]
    sc = jnp.dot(feat_ref[...], wsc_ref[...],
                 preferred_element_type=jnp.float32) + bsc_ref[...]
    out_ref[...] = _leaky(x3 + sc)


def _stage_c(g, features, wkp_flat, b_conv, W2, b2, Wsc, bsc):
    full = lambda shape: pl.BlockSpec(shape, lambda i: (0, 0))
    return pl.pallas_call(
        _block_kernel,
        grid=(NBLK,),
        in_specs=[
            pl.BlockSpec((BN * NEIGH, TW), lambda i: (i, 0)),
            pl.BlockSpec((BN, IN_DIM), lambda i: (i, 0)),
            full((K * MID, MID)),
            full((1, MID)),
            full((MID, OUT_DIM)),
            full((1, OUT_DIM)),
            full((IN_DIM, OUT_DIM)),
            full((1, OUT_DIM)),
        ],
        out_specs=pl.BlockSpec((BN, OUT_DIM), lambda i: (i, 0)),
        out_shape=jax.ShapeDtypeStruct((N, OUT_DIM), jnp.float32),
    )(g, features, wkp_flat, b_conv, W2, b2, Wsc, bsc)


def kernel(features, points, neighbors, W1, b1, kernel_points, Wkp, b_conv,
           W2, b2, Wsc, bsc):
    kp_rows = jnp.zeros((8, TW), jnp.float32)
    for j in range(3):
        kp_rows = kp_rows.at[0, 16 * j:16 * j + K].set(kernel_points[:, j])
    table = _build_table(features, points, kp_rows, W1, b1)
    g = _sc_gather(table, neighbors.reshape(_ROWS))
    wkp_flat = Wkp.reshape(K * MID, MID)
    return _stage_c(g, features, wkp_flat, b_conv.reshape(1, MID), W2,
                    b2.reshape(1, OUT_DIM), Wsc, bsc.reshape(1, OUT_DIM))


# speedup vs baseline: 1.5388x; 1.5388x over previous
import functools

import jax
import jax.numpy as jnp
from jax import lax
from jax.experimental import pallas as pl
from jax.experimental.pallas import tpu as pltpu
from jax.experimental.pallas import tpu_sc as plsc

N = 10000
NEIGH = 32
IN_DIM = 128
OUT_DIM = 256
MID = 64
K = 15
KP_EXTENT = 1.2
TW = 128
BN = 400
NBLK = N // BN

_SC = plsc.get_sparse_core_info()
_NC = _SC.num_cores
_NS = _SC.num_subcores
_NW = _NC * _NS
_ROWS = N * NEIGH
_RPW = _ROWS // _NW
_CHUNK = 400


def _leaky(x):
    return jnp.where(x >= 0, x, 0.1 * x)


def _table_kernel(feat_ref, pts_ref, w1_ref, b1_ref, out_ref):
    x = jnp.dot(feat_ref[...], w1_ref[...], preferred_element_type=jnp.float32)
    x = _leaky(x + b1_ref[...])
    pts = pts_ref[...]
    pn2 = jnp.sum(pts * pts, axis=1, keepdims=True)
    pad = jnp.zeros((N, TW - MID - 4), dtype=jnp.float32)
    out_ref[...] = jnp.concatenate([x, pts, pn2, pad], axis=1)


def _build_table(features, points, W1, b1):
    return pl.pallas_call(
        _table_kernel,
        out_shape=jax.ShapeDtypeStruct((N, TW), jnp.float32),
    )(features, points, W1, b1.reshape(1, MID))


def _sc_gather(table, idx_flat):
    mesh = plsc.VectorSubcoreMesh(core_axis_name="c", subcore_axis_name="s")

    @functools.partial(
        pl.kernel,
        mesh=mesh,
        out_type=jax.ShapeDtypeStruct((_ROWS, TW), jnp.float32),
        scratch_types=[
            pltpu.VMEM((_CHUNK,), jnp.int32),
            pltpu.VMEM((_CHUNK, TW), jnp.float32),
            pltpu.SemaphoreType.DMA,
        ],
    )
    def gather_k(table_hbm, idx_hbm, out_hbm, idx_v, rows_v, sem):
        wid = lax.axis_index("s") * _NC + lax.axis_index("c")
        base = wid * _RPW

        def body(i, _):
            off = base + i * _CHUNK
            pltpu.sync_copy(idx_hbm.at[pl.ds(off, _CHUNK)], idx_v)
            pltpu.async_copy(table_hbm.at[idx_v], rows_v, sem).wait()
            pltpu.sync_copy(rows_v, out_hbm.at[pl.ds(off, _CHUNK)])
            return 0

        lax.fori_loop(0, _RPW // _CHUNK, body, 0)

    return gather_k(table, idx_flat)


def _block_kernel(g_ref, feat_ref, pts_ref, kp2t_ref, kpn2_ref, wkp_ref,
                  bconv_ref, w2_ref, b2_ref, wsc_ref, bsc_ref, out_ref):
    M = BN * NEIGH
    g = g_ref[...]
    nx = g[:, :MID]
    pn = g[:, MID:MID + 3]
    q = pts_ref[...]

    c = pn.reshape(BN, NEIGH, 3) - q[:, None, :]
    cf = c.reshape(M, 3)
    cn2 = jnp.sum(cf * cf, axis=1, keepdims=True)
    d2 = cn2 - jnp.dot(cf, kp2t_ref[...],
                       preferred_element_type=jnp.float32) + kpn2_ref[...]
    d2 = jnp.maximum(d2, 0.0)
    dist = d2 * lax.rsqrt(d2 + 1e-36)
    w = jnp.maximum(1.0 - dist * (1.0 / KP_EXTENT), 0.0)

    wr = w.reshape(BN, NEIGH, K)
    nxr = nx.reshape(BN, NEIGH, MID)
    agg = lax.dot_general(wr, nxr, (((1,), (1,)), ((0,), (0,))),
                          preferred_element_type=jnp.float32)
    agg = agg.reshape(BN, K * MID)

    xkp = jnp.dot(agg, wkp_ref[...], preferred_element_type=jnp.float32)
    x2 = _leaky(xkp + bconv_ref[...])
    x3 = jnp.dot(x2, w2_ref[...], preferred_element_type=jnp.float32) + b2_ref[...]
    sc = jnp.dot(feat_ref[...], wsc_ref[...],
                 preferred_element_type=jnp.float32) + bsc_ref[...]
    out_ref[...] = _leaky(x3 + sc)


def _stage_c(g, features, points, kp2t, kpn2, wkp_flat, b_conv, W2, b2, Wsc, bsc):
    full = lambda shape: pl.BlockSpec(shape, lambda i: (0, 0))
    return pl.pallas_call(
        _block_kernel,
        grid=(NBLK,),
        in_specs=[
            pl.BlockSpec((BN * NEIGH, TW), lambda i: (i, 0)),
            pl.BlockSpec((BN, IN_DIM), lambda i: (i, 0)),
            pl.BlockSpec((BN, 3), lambda i: (i, 0)),
            full((3, K)),
            full((1, K)),
            full((K * MID, MID)),
            full((1, MID)),
            full((MID, OUT_DIM)),
            full((1, OUT_DIM)),
            full((IN_DIM, OUT_DIM)),
            full((1, OUT_DIM)),
        ],
        out_specs=pl.BlockSpec((BN, OUT_DIM), lambda i: (i, 0)),
        out_shape=jax.ShapeDtypeStruct((N, OUT_DIM), jnp.float32),
    )(g, features, points, kp2t, kpn2, wkp_flat, b_conv, W2, b2, Wsc, bsc)


def kernel(features, points, neighbors, W1, b1, kernel_points, Wkp, b_conv,
           W2, b2, Wsc, bsc):
    table = _build_table(features, points, W1, b1)
    g = _sc_gather(table, neighbors.reshape(_ROWS))
    kp2t = 2.0 * kernel_points.T
    kpn2 = jnp.sum(kernel_points * kernel_points, axis=1).reshape(1, K)
    wkp_flat = Wkp.reshape(K * MID, MID)
    return _stage_c(g, features, points, kp2t, kpn2, wkp_flat,
                    b_conv.reshape(1, MID), W2, b2.reshape(1, OUT_DIM),
                    Wsc, bsc.reshape(1, OUT_DIM))


# speedup vs baseline: 1.7929x; 1.1651x over previous
import functools

import jax
import jax.numpy as jnp
from jax import lax
from jax.experimental import pallas as pl
from jax.experimental.pallas import tpu as pltpu
from jax.experimental.pallas import tpu_sc as plsc

N = 10000
NEIGH = 32
IN_DIM = 128
OUT_DIM = 256
MID = 64
K = 15
KP_EXTENT = 1.2
TW = 128
BN = 400
NBLK = N // BN

_SC = plsc.get_sparse_core_info()
_NC = _SC.num_cores
_NS = _SC.num_subcores
_NW = _NC * _NS
_ROWS = N * NEIGH
_RPW = _ROWS // _NW
_CHUNK = 400


def _leaky(x):
    return jnp.where(x >= 0, x, 0.1 * x)


def _table_kernel(feat_ref, pts_ref, w1_ref, b1_ref, out_ref):
    x = jnp.dot(feat_ref[...], w1_ref[...], preferred_element_type=jnp.float32)
    x = _leaky(x + b1_ref[...])
    pts = pts_ref[...]
    pn2 = jnp.sum(pts * pts, axis=1, keepdims=True)
    pad = jnp.zeros((N, TW - MID - 4), dtype=jnp.float32)
    out_ref[...] = jnp.concatenate([x, pts, pn2, pad], axis=1)


def _build_table(features, points, W1, b1):
    return pl.pallas_call(
        _table_kernel,
        out_shape=jax.ShapeDtypeStruct((N, TW), jnp.float32),
    )(features, points, W1, b1.reshape(1, MID))


def _sc_gather(table, idx_flat, nrows):
    mesh = plsc.VectorSubcoreMesh(core_axis_name="c", subcore_axis_name="s")
    rpw = nrows // _NW

    @functools.partial(
        pl.kernel,
        mesh=mesh,
        out_type=jax.ShapeDtypeStruct((nrows, TW), jnp.float32),
        scratch_types=[
            pltpu.VMEM((_CHUNK,), jnp.int32),
            pltpu.VMEM((_CHUNK, TW), jnp.float32),
            pltpu.SemaphoreType.DMA,
        ],
    )
    def gather_k(table_hbm, idx_hbm, out_hbm, idx_v, rows_v, sem):
        wid = lax.axis_index("s") * _NC + lax.axis_index("c")
        base = wid * rpw

        def body(i, _):
            off = base + i * _CHUNK
            pltpu.sync_copy(idx_hbm.at[pl.ds(off, _CHUNK)], idx_v)
            pltpu.async_copy(table_hbm.at[idx_v], rows_v, sem).wait()
            pltpu.sync_copy(rows_v, out_hbm.at[pl.ds(off, _CHUNK)])
            return 0

        lax.fori_loop(0, rpw // _CHUNK, body, 0)

    return gather_k(table, idx_flat)


def _block_kernel(g_ref, feat_ref, pts_ref, kp2t_ref, kpn2_ref, wkp_ref,
                  bconv_ref, w2_ref, b2_ref, wsc_ref, bsc_ref, out_ref):
    M = BN * NEIGH
    g = g_ref[...]
    nx = g[:, :MID]
    pn = g[:, MID:MID + 3]
    q = pts_ref[...]

    c = pn.reshape(BN, NEIGH, 3) - q[:, None, :]
    cf = c.reshape(M, 3)
    cn2 = jnp.sum(cf * cf, axis=1, keepdims=True)
    d2 = cn2 - jnp.dot(cf, kp2t_ref[...],
                       preferred_element_type=jnp.float32) + kpn2_ref[...]
    d2 = jnp.maximum(d2, 0.0)
    dist = d2 * lax.rsqrt(d2 + 1e-36)
    w = jnp.maximum(1.0 - dist * (1.0 / KP_EXTENT), 0.0)

    wr = w.reshape(BN, NEIGH, K)
    nxr = nx.reshape(BN, NEIGH, MID)
    agg = lax.dot_general(wr, nxr, (((1,), (1,)), ((0,), (0,))),
                          preferred_element_type=jnp.float32)
    agg = agg.reshape(BN, K * MID)

    xkp = jnp.dot(agg, wkp_ref[...], preferred_element_type=jnp.float32)
    x2 = _leaky(xkp + bconv_ref[...])
    x3 = jnp.dot(x2, w2_ref[...], preferred_element_type=jnp.float32) + b2_ref[...]
    sc = jnp.dot(feat_ref[...], wsc_ref[...],
                 preferred_element_type=jnp.float32) + bsc_ref[...]
    out_ref[...] = _leaky(x3 + sc)


def _stage_c(g, features, points, kp2t, kpn2, wkp_flat, b_conv, W2, b2, Wsc,
             bsc, npts):
    full = lambda shape: pl.BlockSpec(shape, lambda i: (0, 0))
    return pl.pallas_call(
        _block_kernel,
        grid=(npts // BN,),
        in_specs=[
            pl.BlockSpec((BN * NEIGH, TW), lambda i: (i, 0)),
            pl.BlockSpec((BN, IN_DIM), lambda i: (i, 0)),
            pl.BlockSpec((BN, 3), lambda i: (i, 0)),
            full((3, K)),
            full((1, K)),
            full((K * MID, MID)),
            full((1, MID)),
            full((MID, OUT_DIM)),
            full((1, OUT_DIM)),
            full((IN_DIM, OUT_DIM)),
            full((1, OUT_DIM)),
        ],
        out_specs=pl.BlockSpec((BN, OUT_DIM), lambda i: (i, 0)),
        out_shape=jax.ShapeDtypeStruct((npts, OUT_DIM), jnp.float32),
    )(g, features, points, kp2t, kpn2, wkp_flat, b_conv, W2, b2, Wsc, bsc)


def kernel(features, points, neighbors, W1, b1, kernel_points, Wkp, b_conv,
           W2, b2, Wsc, bsc):
    table = _build_table(features, points, W1, b1)
    kp2t = 2.0 * kernel_points.T
    kpn2 = jnp.sum(kernel_points * kernel_points, axis=1).reshape(1, K)
    wkp_flat = Wkp.reshape(K * MID, MID)
    idx = neighbors.reshape(_ROWS)
    NA = 4800
    outs = []
    for lo, hi in ((0, NA), (NA, N)):
        g = _sc_gather(table, idx[lo * NEIGH:hi * NEIGH], (hi - lo) * NEIGH)
        outs.append(_stage_c(g, features[lo:hi], points[lo:hi], kp2t, kpn2,
                             wkp_flat, b_conv.reshape(1, MID), W2,
                             b2.reshape(1, OUT_DIM), Wsc,
                             bsc.reshape(1, OUT_DIM), hi - lo))
    return jnp.concatenate(outs, axis=0)


# speedup vs baseline: 1.8580x; 1.0363x over previous
import functools

import jax
import jax.numpy as jnp
from jax import lax
from jax.experimental import pallas as pl
from jax.experimental.pallas import tpu as pltpu
from jax.experimental.pallas import tpu_sc as plsc

N = 10000
NEIGH = 32
IN_DIM = 128
OUT_DIM = 256
MID = 64
K = 15
KP_EXTENT = 1.2
TW = 128
BN = 400
NBLK = N // BN

_SC = plsc.get_sparse_core_info()
_NC = _SC.num_cores
_NS = _SC.num_subcores
_NW = _NC * _NS
_ROWS = N * NEIGH
_RPW = _ROWS // _NW
_CHUNK = 400


def _leaky(x):
    return jnp.where(x >= 0, x, 0.1 * x)


def _table_kernel(feat_ref, pts_ref, w1_ref, b1_ref, out_ref):
    x = jnp.dot(feat_ref[...], w1_ref[...], preferred_element_type=jnp.float32)
    x = _leaky(x + b1_ref[...])
    pts = pts_ref[...]
    pn2 = jnp.sum(pts * pts, axis=1, keepdims=True)
    pad = jnp.zeros((N, TW - MID - 4), dtype=jnp.float32)
    out_ref[...] = jnp.concatenate([x, pts, pn2, pad], axis=1)


def _build_table(features, points, W1, b1):
    return pl.pallas_call(
        _table_kernel,
        out_shape=jax.ShapeDtypeStruct((N, TW), jnp.float32),
    )(features, points, W1, b1.reshape(1, MID))


def _sc_gather(table, idx_flat, nrows):
    mesh = plsc.VectorSubcoreMesh(core_axis_name="c", subcore_axis_name="s")
    rpw = nrows // _NW

    @functools.partial(
        pl.kernel,
        mesh=mesh,
        out_type=jax.ShapeDtypeStruct((nrows, TW), jnp.float32),
        scratch_types=[
            pltpu.VMEM((_CHUNK,), jnp.int32),
            pltpu.VMEM((_CHUNK, TW), jnp.float32),
            pltpu.SemaphoreType.DMA,
        ],
    )
    def gather_k(table_hbm, idx_hbm, out_hbm, idx_v, rows_v, sem):
        wid = lax.axis_index("s") * _NC + lax.axis_index("c")
        base = wid * rpw

        def body(i, _):
            off = base + i * _CHUNK
            pltpu.sync_copy(idx_hbm.at[pl.ds(off, _CHUNK)], idx_v)
            pltpu.async_copy(table_hbm.at[idx_v], rows_v, sem).wait()
            pltpu.sync_copy(rows_v, out_hbm.at[pl.ds(off, _CHUNK)])
            return 0

        lax.fori_loop(0, rpw // _CHUNK, body, 0)

    return gather_k(table, idx_flat)


def _block_kernel(g_ref, feat_ref, pts_ref, kp2t_ref, kpn2_ref, wkp_ref,
                  bconv_ref, w2_ref, b2_ref, wsc_ref, bsc_ref, out_ref):
    M = BN * NEIGH
    g = g_ref[...]
    nx = g[:, :MID]
    pn = g[:, MID:MID + 3]
    q = pts_ref[...]

    c = pn.reshape(BN, NEIGH, 3) - q[:, None, :]
    cf = c.reshape(M, 3)
    cn2 = jnp.sum(cf * cf, axis=1, keepdims=True)
    d2 = cn2 - jnp.dot(cf, kp2t_ref[...],
                       preferred_element_type=jnp.float32) + kpn2_ref[...]
    d2 = jnp.maximum(d2, 0.0)
    dist = d2 * lax.rsqrt(d2 + 1e-36)
    w = jnp.maximum(1.0 - dist * (1.0 / KP_EXTENT), 0.0)

    wr = w.reshape(BN, NEIGH, K)
    nxr = nx.reshape(BN, NEIGH, MID)
    agg = lax.dot_general(wr, nxr, (((1,), (1,)), ((0,), (0,))),
                          preferred_element_type=jnp.float32)
    agg = agg.reshape(BN, K * MID)

    xkp = jnp.dot(agg, wkp_ref[...], preferred_element_type=jnp.float32)
    x2 = _leaky(xkp + bconv_ref[...])
    x3 = jnp.dot(x2, w2_ref[...], preferred_element_type=jnp.float32) + b2_ref[...]
    sc = jnp.dot(feat_ref[...], wsc_ref[...],
                 preferred_element_type=jnp.float32) + bsc_ref[...]
    out_ref[...] = _leaky(x3 + sc)


def _stage_c(g, features, points, kp2t, kpn2, wkp_flat, b_conv, W2, b2, Wsc,
             bsc, npts):
    full = lambda shape: pl.BlockSpec(shape, lambda i: (0, 0))
    return pl.pallas_call(
        _block_kernel,
        grid=(npts // BN,),
        in_specs=[
            pl.BlockSpec((BN * NEIGH, TW), lambda i: (i, 0)),
            pl.BlockSpec((BN, IN_DIM), lambda i: (i, 0)),
            pl.BlockSpec((BN, 3), lambda i: (i, 0)),
            full((3, K)),
            full((1, K)),
            full((K * MID, MID)),
            full((1, MID)),
            full((MID, OUT_DIM)),
            full((1, OUT_DIM)),
            full((IN_DIM, OUT_DIM)),
            full((1, OUT_DIM)),
        ],
        out_specs=pl.BlockSpec((BN, OUT_DIM), lambda i: (i, 0)),
        out_shape=jax.ShapeDtypeStruct((npts, OUT_DIM), jnp.float32),
    )(g, features, points, kp2t, kpn2, wkp_flat, b_conv, W2, b2, Wsc, bsc)


def kernel(features, points, neighbors, W1, b1, kernel_points, Wkp, b_conv,
           W2, b2, Wsc, bsc):
    table = _build_table(features, points, W1, b1)
    kp2t = 2.0 * kernel_points.T
    kpn2 = jnp.sum(kernel_points * kernel_points, axis=1).reshape(1, K)
    wkp_flat = Wkp.reshape(K * MID, MID)
    idx = neighbors.reshape(_ROWS)
    outs = []
    for lo, hi in ((0, 2400), (2400, 4800), (4800, 7200), (7200, N)):
        g = _sc_gather(table, idx[lo * NEIGH:hi * NEIGH], (hi - lo) * NEIGH)
        outs.append(_stage_c(g, features[lo:hi], points[lo:hi], kp2t, kpn2,
                             wkp_flat, b_conv.reshape(1, MID), W2,
                             b2.reshape(1, OUT_DIM), Wsc,
                             bsc.reshape(1, OUT_DIM), hi - lo))
    return jnp.concatenate(outs, axis=0)
